# baseline (speedup 1.0000x reference)
import functools

import jax
import jax.numpy as jnp
from jax.experimental import pallas as pl
from jax.experimental.pallas import tpu as pltpu


def _fused_body(x_ref, wdisc_ref, mapping_ref, wb_ref, bb_ref, wdown_hbm,
                wup_hbm, out_ref, xres, acc, aidx_s, wd_buf, wu_buf,
                sem_d, sem_u, *, B, Tn, TT, DD, RR):
    p = pl.program_id(0)
    t = pl.program_id(1)

    @pl.when(p < B)
    def _route():
        @pl.when(t == 0)
        def _():
            acc[...] = jnp.zeros_like(acc)

        xb = x_ref[...]
        xres[p % 2, pl.ds(t * TT, TT), :] = xb
        proj = jnp.dot(xb, wdisc_ref[...], preferred_element_type=jnp.float32)
        acc[...] += jnp.sum(proj * proj, axis=0, keepdims=True)

        @pl.when(t == Tn - 1)
        def _():
            e = mapping_ref[jnp.argmin(acc[0], axis=0)]
            aidx_s[p % 2] = e
            pltpu.make_async_copy(wdown_hbm.at[pl.ds(e * DD, DD)],
                                  wd_buf.at[p % 2], sem_d).start()
            pltpu.make_async_copy(wup_hbm.at[pl.ds(e * RR, RR)],
                                  wu_buf.at[p % 2], sem_u).start()

    @pl.when(p >= 1)
    def _main():
        s = (p - 1) % 2

        @pl.when(t == 0)
        def _():
            e = aidx_s[s]
            pltpu.make_async_copy(wdown_hbm.at[pl.ds(e * DD, DD)],
                                  wd_buf.at[s], sem_d).wait()
            pltpu.make_async_copy(wup_hbm.at[pl.ds(e * RR, RR)],
                                  wu_buf.at[s], sem_u).wait()

        xb = xres[s, pl.ds(t * TT, TT), :]
        base = jnp.dot(xb, wb_ref[...], preferred_element_type=jnp.float32)
        base = base + bb_ref[...]
        h = jnp.maximum(jnp.dot(xb, wd_buf[s], preferred_element_type=jnp.float32), 0.0)
        out_ref[...] = base + jnp.dot(h, wu_buf[s], preferred_element_type=jnp.float32)


@jax.jit
def _run(x, Wb, bb, Wdisc, Wdown, Wup, mapping):
    B, T, D = x.shape
    E, _, R = Wdown.shape
    TT = 1024
    Tn = T // TT

    body = functools.partial(_fused_body, B=B, Tn=Tn, TT=TT, DD=D, RR=R)

    out = pl.pallas_call(
        body,
        grid=(B + 1, Tn),
        in_specs=[
            pl.BlockSpec(
                (TT, D),
                lambda p, t: (jnp.where(p < B, p * Tn + t, B * Tn - 1), 0)),
            pl.BlockSpec((D, E), lambda p, t: (0, 0)),
            pl.BlockSpec(memory_space=pltpu.SMEM),
            pl.BlockSpec((D, D), lambda p, t: (0, 0)),
            pl.BlockSpec((1, D), lambda p, t: (0, 0)),
            pl.BlockSpec(memory_space=pl.ANY),
            pl.BlockSpec(memory_space=pl.ANY),
        ],
        out_specs=pl.BlockSpec(
            (TT, D),
            lambda p, t: (jnp.maximum(p - 1, 0) * Tn +
                          jnp.where(p >= 1, t, 0), 0)),
        out_shape=jax.ShapeDtypeStruct((B * T, D), jnp.float32),
        scratch_shapes=[
            pltpu.VMEM((2, T, D), jnp.float32),
            pltpu.VMEM((1, E), jnp.float32),
            pltpu.SMEM((2,), jnp.int32),
            pltpu.VMEM((2, D, R), jnp.float32),
            pltpu.VMEM((2, R, D), jnp.float32),
            pltpu.SemaphoreType.DMA,
            pltpu.SemaphoreType.DMA,
        ],
        compiler_params=pltpu.CompilerParams(
            dimension_semantics=("arbitrary", "arbitrary")),
    )(x.reshape(B * T, D), Wdisc.T, mapping, Wb, bb.reshape(1, D),
      Wdown.reshape(E * D, R), Wup.reshape(E * R, D))
    return out.reshape(B, T, D)


def kernel(x, Wb, bb, Wdisc, Wdown, Wup, mapping):
    return _run(x, Wb, bb, Wdisc, Wdown, Wup, mapping)


# speedup vs baseline: 1.0565x; 1.0565x over previous
import functools

import jax
import jax.numpy as jnp
from jax.experimental import pallas as pl
from jax.experimental.pallas import tpu as pltpu


def _fused_body(x_ref, wdisc_ref, mapping_ref, wb_ref, bb_ref, wdown_hbm,
                wup_hbm, out_ref, xres, acc, aidx_s, wd_buf, wu_buf,
                sem_d, sem_u, *, B, Tn, TT):
    p = pl.program_id(0)
    t = pl.program_id(1)

    @pl.when(p < B)
    def _route():
        @pl.when(t == 0)
        def _():
            acc[...] = jnp.zeros_like(acc)

        xb = x_ref[...]
        xres[p % 2, pl.ds(t * TT, TT), :] = xb
        proj = jnp.dot(xb, wdisc_ref[...], preferred_element_type=jnp.float32)
        acc[...] += jnp.sum(proj * proj, axis=0, keepdims=True)

        @pl.when(t == Tn - 1)
        def _():
            e = mapping_ref[jnp.argmin(acc[0], axis=0)]
            aidx_s[p % 2] = e
            pltpu.make_async_copy(wdown_hbm.at[e], wd_buf.at[p % 2], sem_d).start()
            pltpu.make_async_copy(wup_hbm.at[e], wu_buf.at[p % 2], sem_u).start()

    @pl.when(p >= 1)
    def _main():
        s = (p - 1) % 2

        @pl.when(t == 0)
        def _():
            e = aidx_s[s]
            pltpu.make_async_copy(wdown_hbm.at[e], wd_buf.at[s], sem_d).wait()
            pltpu.make_async_copy(wup_hbm.at[e], wu_buf.at[s], sem_u).wait()

        xb = xres[s, pl.ds(t * TT, TT), :]
        base = jnp.dot(xb, wb_ref[...], preferred_element_type=jnp.float32)
        base = base + bb_ref[...]
        h = jnp.maximum(jnp.dot(xb, wd_buf[s], preferred_element_type=jnp.float32), 0.0)
        out_ref[...] = base + jnp.dot(h, wu_buf[s], preferred_element_type=jnp.float32)


@jax.jit
def _run(x, Wb, bb, Wdisc, Wdown, Wup, mapping):
    B, T, D = x.shape
    E, _, R = Wdown.shape
    TT = 1024
    Tn = T // TT

    body = functools.partial(_fused_body, B=B, Tn=Tn, TT=TT)

    out = pl.pallas_call(
        body,
        grid=(B + 1, Tn),
        in_specs=[
            pl.BlockSpec(
                (TT, D),
                lambda p, t: (jnp.where(p < B, p * Tn + t, B * Tn - 1), 0)),
            pl.BlockSpec((D, E), lambda p, t: (0, 0)),
            pl.BlockSpec(memory_space=pltpu.SMEM),
            pl.BlockSpec((D, D), lambda p, t: (0, 0)),
            pl.BlockSpec((1, D), lambda p, t: (0, 0)),
            pl.BlockSpec(memory_space=pl.ANY),
            pl.BlockSpec(memory_space=pl.ANY),
        ],
        out_specs=pl.BlockSpec(
            (TT, D),
            lambda p, t: (jnp.maximum(p - 1, 0) * Tn +
                          jnp.where(p >= 1, t, 0), 0)),
        out_shape=jax.ShapeDtypeStruct((B * T, D), jnp.float32),
        scratch_shapes=[
            pltpu.VMEM((2, T, D), jnp.float32),
            pltpu.VMEM((1, E), jnp.float32),
            pltpu.SMEM((2,), jnp.int32),
            pltpu.VMEM((2, D, R), jnp.float32),
            pltpu.VMEM((2, R, D), jnp.float32),
            pltpu.SemaphoreType.DMA,
            pltpu.SemaphoreType.DMA,
        ],
        compiler_params=pltpu.CompilerParams(
            dimension_semantics=("arbitrary", "arbitrary")),
    )(x.reshape(B * T, D), Wdisc.T, mapping, Wb, bb.reshape(1, D), Wdown, Wup)
    return out.reshape(B, T, D)


def kernel(x, Wb, bb, Wdisc, Wdown, Wup, mapping):
    return _run(x, Wb, bb, Wdisc, Wdown, Wup, mapping)
